# Initial kernel scaffold; baseline (speedup 1.0000x reference)
import jax
import jax.numpy as jnp
from jax.experimental import pallas as pl


def kernel(x, edge_index, edge_attr, Wm1, bm1, Wu1, bu1, Wm2, bm2, Wu2, bu2, Wm3, bm3, Wu3, bu3, Wm4, bm4, Wu4, bu4, Wm5, bm5, Wu5, bu5):
    raise NotImplementedError("write your pallas kernel here")


# speedup vs baseline: 1.4004x; 1.4004x over previous
import functools

import jax
import jax.numpy as jnp
from jax import lax
from jax.experimental import pallas as pl
from jax.experimental.pallas import tpu as pltpu
from jax.experimental.pallas import tpu_sc as plsc

NC = 2
NS = 16
CH = 128
LANES = 16


def _mm(parts, weights, bias, relu, bm):
    m = parts[0].shape[0]
    n = len(parts)
    has_b = bias is not None

    def body(*refs):
        arefs = refs[:n]
        wrefs = refs[n:2 * n]
        acc = None
        for a, w in zip(arefs, wrefs):
            d = jnp.dot(a[...], w[...], preferred_element_type=jnp.float32)
            acc = d if acc is None else acc + d
        if has_b:
            acc = acc + refs[2 * n][...]
        if relu:
            acc = jnp.maximum(acc, 0.0)
        refs[-1][...] = acc

    in_specs = [pl.BlockSpec((bm, p.shape[1]), lambda j: (j, 0)) for p in parts]
    in_specs += [pl.BlockSpec(w.shape, lambda j: (0, 0)) for w in weights]
    ops = [*parts, *weights]
    if has_b:
        in_specs.append(pl.BlockSpec((1, 128), lambda j: (0, 0)))
        ops.append(bias.reshape(1, 128))
    return pl.pallas_call(
        body,
        grid=(m // bm,),
        in_specs=in_specs,
        out_specs=pl.BlockSpec((bm, 128), lambda j: (j, 0)),
        out_shape=jax.ShapeDtypeStruct((m, 128), jnp.float32),
    )(*ops)


@functools.lru_cache(maxsize=None)
def _make_sc_edge(npad, cps):
    half = npad // NC
    rtot = half + NS * 64
    zero_copies = rtot // NS // 64
    out_rows_per_sub = half // NS
    out_copies = out_rows_per_sub // 64
    mesh = plsc.VectorSubcoreMesh(core_axis_name="c", subcore_axis_name="s")

    @functools.partial(
        pl.kernel,
        mesh=mesh,
        out_type=jax.ShapeDtypeStruct((npad, 128), jnp.float32),
        scratch_types=[
            pltpu.VMEM((cps, CH), jnp.int32),
            pltpu.VMEM((cps, CH), jnp.int32),
            pltpu.VMEM((CH,), jnp.int32),
            pltpu.VMEM((CH, 128), jnp.float32),
            pltpu.VMEM((CH, 128), jnp.float32),
            pltpu.VMEM_SHARED((rtot, 128), jnp.float32),
            pltpu.SemaphoreType.DMA,
        ],
    )
    def sc_edge(t_hbm, ep_hbm, src_hbm, dst_hbm, out_hbm,
                src_v, dst_v, dstm_v, rows_v, ep_v, agg_sh, sem):
        cid = lax.axis_index("c")
        sid = lax.axis_index("s")
        lo = cid * half

        def zrow(i, carry):
            for g in range(128 // LANES):
                rows_v[i, pl.ds(g * LANES, LANES)] = jnp.zeros((LANES,), jnp.float32)
            return carry

        lax.fori_loop(0, CH, zrow, 0)
        for k in range(zero_copies):
            pltpu.sync_copy(rows_v.at[pl.ds(0, 64)],
                            agg_sh.at[pl.ds(sid * (rtot // NS) + k * 64, 64)])
        plsc.subcore_barrier()

        base = sid * cps
        pltpu.sync_copy(src_hbm.at[pl.ds(base, cps)], src_v)
        pltpu.sync_copy(dst_hbm.at[pl.ds(base, cps)], dst_v)

        def chunk(j, carry):
            pltpu.async_copy(t_hbm.at[src_v.at[j]], rows_v, sem).wait()
            pltpu.sync_copy(ep_hbm.at[pl.ds((base + j) * CH, CH)], ep_v)

            for g in range(CH // LANES):
                s = pl.ds(g * LANES, LANES)
                dv = dst_v[j, s] - lo
                ok = (dv >= 0) & (dv < half)
                dstm_v[s] = jnp.where(ok, dv, half)

            def row(i, c2):
                for g in range(128 // LANES):
                    s = pl.ds(g * LANES, LANES)
                    rows_v[i, s] = jnp.maximum(rows_v[i, s] + ep_v[i, s], 0.0)
                return c2

            lax.fori_loop(0, CH, row, 0)
            pltpu.sync_copy(rows_v, agg_sh.at[dstm_v], add=True)
            return carry

        lax.fori_loop(0, cps, chunk, 0)
        plsc.subcore_barrier()

        for k in range(out_copies):
            r = sid * out_rows_per_sub + k * 64
            pltpu.sync_copy(agg_sh.at[pl.ds(r, 64)], rows_v.at[pl.ds(0, 64)])
            pltpu.sync_copy(rows_v.at[pl.ds(0, 64)], out_hbm.at[pl.ds(lo + r, 64)])

    return sc_edge


def kernel(x, edge_index, edge_attr,
           Wm1, bm1, Wu1, bu1,
           Wm2, bm2, Wu2, bu2,
           Wm3, bm3, Wu3, bu3,
           Wm4, bm4, Wu4, bu4,
           Wm5, bm5, Wu5, bu5):
    n, d = x.shape
    e = edge_index.shape[1]
    de = edge_attr.shape[1]

    npad = ((n + NS * CH - 1) // (NS * CH)) * (NS * CH)
    cps = -(-((e + NS * CH - 1) // (NS * CH)) // 8) * 8
    ep_total = NS * cps * CH

    src = edge_index[0].astype(jnp.int32)
    dst = edge_index[1].astype(jnp.int32)
    pad_e = ep_total - e
    src_p = jnp.concatenate([src, jnp.zeros((pad_e,), jnp.int32)]).reshape(NS * cps, CH)
    dst_p = jnp.concatenate([dst, jnp.full((pad_e,), n, jnp.int32)]).reshape(NS * cps, CH)
    ea_p = jnp.concatenate([edge_attr, jnp.zeros((pad_e, de), jnp.float32)], axis=0)
    h0 = jnp.concatenate([x, jnp.zeros((npad - n, d), jnp.float32)], axis=0)

    sc_edge = _make_sc_edge(npad, cps)

    def layer(h_parts, Wm, bm, Wu, bu):
        cin = Wm.shape[0] - de
        nparts = cin // 128
        wmx = [Wm[i * 128:(i + 1) * 128] for i in range(nparts)]
        wme = Wm[cin:]
        t = _mm(h_parts, wmx, None, False, 1024)
        ep = _mm([ea_p], [wme], bm, False, 2048)
        agg = sc_edge(t, ep, src_p, dst_p)
        wuh = [Wu[i * 128:(i + 1) * 128] for i in range(nparts)]
        wua = Wu[cin:]
        return _mm(h_parts + [agg], wuh + [wua], bu, True, 1024)

    x1 = layer([h0], Wm1, bm1, Wu1, bu1)
    x2 = layer([x1], Wm2, bm2, Wu2, bu2)
    xu = layer([x2], Wm3, bm3, Wu3, bu3)
    x4 = layer([xu, x2], Wm4, bm4, Wu4, bu4)
    out = layer([x4], Wm5, bm5, Wu5, bu5)
    return out[:n]
